# baseline (speedup 1.0000x reference)
import functools

import jax
import jax.numpy as jnp
from jax import lax
from jax.experimental import pallas as pl
from jax.experimental.pallas import tpu as pltpu
from jax.experimental.pallas import tpu_sc as plsc

NC = 2
NS = 16
NW = NC * NS
R = 8
A = 4


def _gather_call(b, s, d):
    bw = b // NW
    dh = d // 2
    mesh = plsc.VectorSubcoreMesh(core_axis_name="c", subcore_axis_name="s")
    n_main = s - 2 * A
    assert n_main % R == 0

    @functools.partial(
        pl.kernel,
        mesh=mesh,
        out_type=jax.ShapeDtypeStruct((b, s, d), jnp.float32),
        scratch_types=[
            pltpu.VMEM((s, bw), jnp.int32),
            pltpu.VMEM((R, bw, dh), jnp.float32),
            pltpu.VMEM((R, bw, dh), jnp.float32),
            pltpu.SemaphoreType.DMA((R,)),
            pltpu.SemaphoreType.DMA((R,)),
        ],
        compiler_params=pltpu.CompilerParams(use_tc_tiling_on_sc=False),
    )
    def k(idx_hbm, t1_hbm, t2_hbm, out_hbm, idx_v, rows1, rows2, gsem, wsem):
        wid = lax.axis_index("s") * NC + lax.axis_index("c")
        b0 = wid * bw
        pltpu.sync_copy(idx_hbm.at[:, pl.ds(b0, bw)], idx_v)

        def start_g(t, buf):
            pltpu.async_copy(t1_hbm.at[idx_v.at[t]], rows1.at[buf], gsem.at[buf])
            pltpu.async_copy(t2_hbm.at[idx_v.at[t]], rows2.at[buf], gsem.at[buf])

        def wait_g(buf):
            for rows in (rows1, rows2):
                pltpu.make_async_copy(
                    t1_hbm.at[idx_v.at[0]], rows.at[buf], gsem.at[buf]
                ).wait()

        def start_w(t, buf):
            pltpu.async_copy(
                rows1.at[buf], out_hbm.at[pl.ds(b0, bw), t, pl.ds(0, dh)],
                wsem.at[buf],
            )
            pltpu.async_copy(
                rows2.at[buf], out_hbm.at[pl.ds(b0, bw), t, pl.ds(dh, dh)],
                wsem.at[buf],
            )

        def wait_w(buf):
            for rows in (rows1, rows2):
                pltpu.make_async_copy(
                    rows.at[buf], out_hbm.at[pl.ds(b0, bw), 0, pl.ds(0, dh)],
                    wsem.at[buf],
                ).wait()

        for t in range(A):
            start_g(t, t)
        for t in range(A):
            start_g(t + A, t + A)
            wait_g(t)
            start_w(t, t)

        @pl.loop(0, n_main // R)
        def body(g):
            t0 = A + R * g
            for i in range(R):
                wait_w(i)
                start_g(t0 + A + i, i)
                sl = (A + i) % R
                wait_g(sl)
                start_w(t0 + i, sl)

        for t in range(s - A, s):
            sl = t % R
            wait_g(sl)
            start_w(t, sl)
        for i in range(R):
            wait_w(i)

    return k


def kernel(token_ids, emb):
    b, s = token_ids.shape
    d = emb.shape[1]
    dh = d // 2
    return _gather_call(b, s, d)(token_ids.T, emb[:, :dh], emb[:, dh:])


# speedup vs baseline: 1.6346x; 1.6346x over previous
import functools

import jax
import jax.numpy as jnp
from jax import lax
from jax.experimental import pallas as pl
from jax.experimental.pallas import tpu as pltpu
from jax.experimental.pallas import tpu_sc as plsc

NC = 2
NS = 16
NW = NC * NS
R = 8
A = 4


def _gather_call(b, s, d):
    bw = b // NW
    mesh = plsc.VectorSubcoreMesh(core_axis_name="c", subcore_axis_name="s")
    n_main = s - 2 * A
    assert n_main % R == 0

    @functools.partial(
        pl.kernel,
        mesh=mesh,
        out_type=jax.ShapeDtypeStruct((b, s, d), jnp.float32),
        scratch_types=[
            pltpu.VMEM((s, bw), jnp.int32),
            pltpu.VMEM((R, bw, d), jnp.float32),
            pltpu.SemaphoreType.DMA((R,)),
            pltpu.SemaphoreType.DMA((R,)),
        ],
        compiler_params=pltpu.CompilerParams(use_tc_tiling_on_sc=False),
    )
    def k(idx_hbm, table_hbm, out_hbm, idx_v, rows, gsem, wsem):
        wid = lax.axis_index("s") * NC + lax.axis_index("c")
        b0 = wid * bw
        pltpu.sync_copy(idx_hbm.at[:, pl.ds(b0, bw)], idx_v)

        def start_g(t, buf):
            pltpu.async_copy(table_hbm.at[idx_v.at[t]], rows.at[buf], gsem.at[buf])

        def wait_g(buf):
            pltpu.make_async_copy(
                table_hbm.at[idx_v.at[0]], rows.at[buf], gsem.at[buf]
            ).wait()

        def start_w(t, buf):
            pltpu.async_copy(
                rows.at[buf], out_hbm.at[pl.ds(b0, bw), t], wsem.at[buf]
            )

        def wait_w(buf):
            pltpu.make_async_copy(
                rows.at[buf], out_hbm.at[pl.ds(b0, bw), 0], wsem.at[buf]
            ).wait()

        for t in range(A):
            start_g(t, t)
        for t in range(A):
            start_g(t + A, t + A)
            wait_g(t)
            start_w(t, t)

        @pl.loop(0, n_main // R)
        def body(g):
            t0 = A + R * g
            for i in range(R):
                wait_w(i)
                start_g(t0 + A + i, i)
                sl = (A + i) % R
                wait_g(sl)
                start_w(t0 + i, sl)

        for t in range(s - A, s):
            sl = t % R
            wait_g(sl)
            start_w(t, sl)
        for i in range(R):
            wait_w(i)

    return k


def kernel(token_ids, emb):
    b, s = token_ids.shape
    d = emb.shape[1]
    return _gather_call(b, s, d)(token_ids.T, emb)
